# baseline (speedup 1.0000x reference)
import functools

import jax
import jax.numpy as jnp
from jax import lax
from jax.experimental import pallas as pl
from jax.experimental.pallas import tpu as pltpu
from jax.experimental.pallas import tpu_sc as plsc

BATCH = 16384
NUM_ACTIONS = 1000
NSENS = 16
NGROUPS = 10
LAMBDA_FAIRNESS = 0.01

NW = 32
RPW = BATCH // NW
CHUNK = 16
NCHUNK = RPW // CHUNK


def _sc_partials(x_hbm, g_hbm, a_hbm, sums_hbm, cnt_hbm, xv, gv, av, accv, cntv):
    wid = lax.axis_index("s") * 2 + lax.axis_index("c")
    base = wid * RPW
    pltpu.sync_copy(x_hbm.at[pl.ds(base, RPW), pl.ds(0, 128)], xv)
    pltpu.sync_copy(g_hbm.at[pl.ds(base, RPW)], gv)
    pltpu.sync_copy(a_hbm.at[pl.ds(base, RPW)], av)

    lane = lax.iota(jnp.int32, 16)
    zero16 = jnp.zeros((16,), jnp.float32)
    ones16 = jnp.ones((16,), jnp.float32)
    for r in range(16):
        accv[pl.ds(r * 16, 16)] = zero16
        cntv[pl.ds(r * 16, 16)] = zero16

    def chunk_body(c, carry):
        g16 = gv[pl.ds(c * CHUNK, 16)]
        a16 = av[pl.ds(c * CHUNK, 16)]
        plsc.addupdate_scatter(cntv, [g16 * 16 + lane], ones16)
        plsc.addupdate_scatter(cntv, [(a16 + 2) * 16 + lane], ones16)
        for i in range(CHUNK):
            row = c * CHUNK + i
            rsplat = jnp.full((16,), row, jnp.int32)
            xrow = plsc.load_gather(xv, [rsplat, lane])
            gsp = plsc.load_gather(gv, [rsplat])
            asp = plsc.load_gather(av, [rsplat])
            plsc.addupdate_scatter(accv, [gsp * 16 + lane], xrow)
            plsc.addupdate_scatter(accv, [(asp + 2) * 16 + lane], xrow)
        return carry

    lax.fori_loop(0, NCHUNK, chunk_body, jnp.int32(0))
    pltpu.sync_copy(accv, sums_hbm.at[wid])
    pltpu.sync_copy(cntv, cnt_hbm.at[wid])


def _pairmask():
    rj = lax.broadcasted_iota(jnp.int32, (16, 16), 0)
    ck = lax.broadcasted_iota(jnp.int32, (16, 16), 1)
    same = jnp.logical_or(
        jnp.logical_and(rj < 2, ck < 2),
        jnp.logical_and(jnp.logical_and(rj >= 2, rj < 10),
                        jnp.logical_and(ck >= 2, ck < 10)))
    return jnp.logical_and(rj > ck, same).astype(jnp.float32)


def _epilogue(s_ref, c_ref, out_ref):
    sums = jnp.sum(s_ref[...], axis=0)
    counts = jnp.sum(jnp.sum(c_ref[...], axis=0), axis=1,
                     keepdims=True)
    present = (counts > 0.0).astype(jnp.float32)
    safe = jnp.where(counts > 0.0, counts, 1.0)
    means = sums / safe
    both = lax.dot_general(
        present, present, (((1,), (1,)), ((), ())),
        preferred_element_type=jnp.float32)
    pm = _pairmask()
    ncomp = float(NSENS) * jnp.sum(pm * both)
    total = jnp.float32(0.0)
    for k in range(NGROUPS):
        d = jnp.abs(means - means[k:k + 1, :])
        s = jnp.sum(d, axis=1, keepdims=True)
        total = total + jnp.sum(s * pm[:, k:k + 1] * both[:, k:k + 1])
    result = jnp.where(
        ncomp > 0.0,
        LAMBDA_FAIRNESS * total / jnp.maximum(ncomp, 1.0),
        0.0)
    out_ref[0, 0] = result


@jax.jit
def kernel(action_probs, demo_gender, demo_age):
    mesh = plsc.VectorSubcoreMesh(core_axis_name="c", subcore_axis_name="s")
    sums, cnt = pl.kernel(
        _sc_partials,
        mesh=mesh,
        compiler_params=pltpu.CompilerParams(needs_layout_passes=False),
        out_type=[
            jax.ShapeDtypeStruct((NW, 256), jnp.float32),
            jax.ShapeDtypeStruct((NW, 256), jnp.float32),
        ],
        scratch_types=[
            pltpu.VMEM((RPW, 128), jnp.float32),
            pltpu.VMEM((RPW,), jnp.int32),
            pltpu.VMEM((RPW,), jnp.int32),
            pltpu.VMEM((256,), jnp.float32),
            pltpu.VMEM((256,), jnp.float32),
        ],
    )(action_probs, demo_gender, demo_age)
    out = pl.pallas_call(
        _epilogue,
        out_specs=pl.BlockSpec(memory_space=pltpu.SMEM),
        out_shape=jax.ShapeDtypeStruct((1, 1), jnp.float32),
    )(sums.reshape(NW, 16, 16), cnt.reshape(NW, 16, 16))
    return out[0, 0]


# speedup vs baseline: 1.8729x; 1.8729x over previous
import jax
import jax.numpy as jnp
from jax import lax
from jax.experimental import pallas as pl
from jax.experimental.pallas import tpu as pltpu
from jax.experimental.pallas import tpu_sc as plsc

BATCH = 16384
NUM_ACTIONS = 1000
NSENS = 16
NGROUPS = 10
LAMBDA_FAIRNESS = 0.01

NT = 16
RPT = BATCH // NT
CHUNK = 16
NCHUNK = RPT // CHUNK

PAIRS = [(0, 1)] + [(i, j) for i in range(2, 10) for j in range(i + 1, 10)]


def _sc_body(x_hbm, c_hbm, out_hbm, xv, cv, accv, cntv, stage, outv, shared):
    sid = lax.axis_index("s") + lax.axis_index("c")
    base = sid * RPT
    pltpu.sync_copy(x_hbm.at[pl.ds(base * NSENS, RPT * NSENS)], xv)
    pltpu.sync_copy(c_hbm.at[pl.ds(base, RPT)], cv)

    lane = lax.iota(jnp.int32, 16)
    zero16 = jnp.zeros((16,), jnp.float32)
    ones16 = jnp.ones((16,), jnp.float32)
    for r in range(16):
        accv[pl.ds(r * 16, 16)] = zero16
        cntv[pl.ds(r * 16, 16)] = zero16

    def chunk_body(c, carry):
        c16 = cv[pl.ds(c * CHUNK, 16)]
        goff = (c16 % 2) * 16
        aoff = (c16 // 2 + 2) * 16
        plsc.addupdate_scatter(cntv, [goff + lane], ones16)
        plsc.addupdate_scatter(cntv, [aoff + lane], ones16)
        for i in range(CHUNK):
            row = c * CHUNK + i
            rsplat = jnp.full((16,), row, jnp.int32)
            xrow = xv[pl.ds(row * NSENS, 16)]
            csp = plsc.load_gather(cv, [rsplat])
            plsc.addupdate_scatter(accv, [(csp % 2) * 16 + lane], xrow)
            plsc.addupdate_scatter(accv, [(csp // 2 + 2) * 16 + lane], xrow)
        return carry

    lax.fori_loop(0, NCHUNK, chunk_body, jnp.int32(0))

    pltpu.sync_copy(accv, shared.at[sid, pl.ds(0, 256)])
    pltpu.sync_copy(cntv, shared.at[sid, pl.ds(256, 256)])
    plsc.subcore_barrier()

    @pl.when(sid == 0)
    def _final():
        for r in range(16):
            accv[pl.ds(r * 16, 16)] = zero16
            cntv[pl.ds(r * 16, 16)] = zero16
        for k in range(NT):
            pltpu.sync_copy(shared.at[k], stage)
            for r in range(16):
                accv[pl.ds(r * 16, 16)] += stage[pl.ds(r * 16, 16)]
                cntv[pl.ds(r * 16, 16)] += stage[pl.ds(256 + r * 16, 16)]
        means = []
        pres = []
        for g in range(NGROUPS):
            cnt_gv = jnp.full((16,), jnp.sum(cntv[pl.ds(g * 16, 16)]),
                              jnp.float32)
            pres.append(jnp.where(cnt_gv > 0.0, 1.0, 0.0))
            safe_gv = jnp.where(cnt_gv > 0.0, cnt_gv, 1.0)
            means.append(accv[pl.ds(g * 16, 16)] / safe_gv)
        total_v = zero16
        ncomp_v = zero16
        for (i, j) in PAIRS:
            both_v = pres[i] * pres[j]
            total_v = total_v + both_v * jnp.abs(means[i] - means[j])
            ncomp_v = ncomp_v + both_v
        totf = jnp.full((16,), jnp.sum(total_v), jnp.float32)
        ncf = jnp.full((16,), jnp.sum(ncomp_v), jnp.float32)
        res_v = jnp.where(
            ncf > 0.0,
            LAMBDA_FAIRNESS * totf / jnp.maximum(ncf, 1.0),
            0.0)
        outv[...] = res_v
        pltpu.sync_copy(outv, out_hbm)


@jax.jit
def kernel(action_probs, demo_gender, demo_age):
    xflat = action_probs[:, :NSENS].reshape(-1)
    combo = demo_gender + 2 * demo_age
    mesh = plsc.VectorSubcoreMesh(
        core_axis_name="c", subcore_axis_name="s", num_cores=1)
    out = pl.kernel(
        _sc_body,
        mesh=mesh,
        compiler_params=pltpu.CompilerParams(needs_layout_passes=False),
        out_type=jax.ShapeDtypeStruct((16,), jnp.float32),
        scratch_types=[
            pltpu.VMEM((RPT * NSENS,), jnp.float32),
            pltpu.VMEM((RPT,), jnp.int32),
            pltpu.VMEM((256,), jnp.float32),
            pltpu.VMEM((256,), jnp.float32),
            pltpu.VMEM((512,), jnp.float32),
            pltpu.VMEM((16,), jnp.float32),
            pltpu.VMEM_SHARED((NT, 512), jnp.float32),
        ],
    )(xflat, combo)
    return out[0]


# speedup vs baseline: 4.5895x; 2.4505x over previous
import jax
import jax.numpy as jnp
from jax import lax
from jax.experimental import pallas as pl
from jax.experimental.pallas import tpu as pltpu

BATCH = 16384
NUM_ACTIONS = 1000
NSENS = 16
NGROUPS = 10
LAMBDA_FAIRNESS = 0.01
R = BATCH // 8


def _pairmask():
    rj = lax.broadcasted_iota(jnp.int32, (16, 16), 0)
    ck = lax.broadcasted_iota(jnp.int32, (16, 16), 1)
    same = jnp.logical_or(
        jnp.logical_and(rj < 2, ck < 2),
        jnp.logical_and(jnp.logical_and(rj >= 2, rj < 10),
                        jnp.logical_and(ck >= 2, ck < 10)))
    return jnp.where(jnp.logical_and(rj > ck, same), 1.0, 0.0)


def _body(x_ref, c_ref, out_ref):
    X = x_ref[...]
    C8 = c_ref[...]
    C128 = jnp.concatenate([C8] * 16, axis=1).astype(jnp.int32)
    j128 = lax.broadcasted_iota(jnp.int32, (R, 128), 1)
    g128 = j128 // 8
    is_gender = g128 < 2
    m_bool = jnp.logical_or(
        jnp.logical_and(is_gender, (C128 % 2) == g128),
        jnp.logical_and(
            jnp.logical_and(jnp.logical_not(is_gender), g128 < NGROUPS),
            (C128 // 2) == (g128 - 2)))
    M = jnp.where(m_bool, 1.0, 0.0)

    T = lax.dot_general(M, X, (((0,), (0,)), ((), ())),
                        preferred_element_type=jnp.float32)
    jrow = lax.broadcasted_iota(jnp.int32, (128, 128), 0)
    clane = lax.broadcasted_iota(jnp.int32, (128, 128), 1)
    D = jnp.where((clane // 16) == (jrow % 8), 1.0, 0.0)
    TD = T * D
    fr = lax.broadcasted_iota(jnp.int32, (128, 16), 0)
    fc = lax.broadcasted_iota(jnp.int32, (128, 16), 1)
    F = jnp.where((fr % 16) == fc, 1.0, 0.0)
    S8 = lax.dot_general(TD, F, (((1,), (0,)), ((), ())),
                         preferred_element_type=jnp.float32)
    hr = lax.broadcasted_iota(jnp.int32, (16, 128), 0)
    hc = lax.broadcasted_iota(jnp.int32, (16, 128), 1)
    H = jnp.where((hc // 8) == hr, 1.0, 0.0)
    sums = lax.dot_general(H, S8, (((1,), (0,)), ((), ())),
                           preferred_element_type=jnp.float32)
    cs = lax.dot_general(M, jnp.ones((R, 1), jnp.float32),
                         (((0,), (0,)), ((), ())),
                         preferred_element_type=jnp.float32)
    counts = lax.dot_general(H, cs, (((1,), (0,)), ((), ())),
                             preferred_element_type=jnp.float32)

    present = jnp.where(counts > 0.0, 1.0, 0.0)
    safe = jnp.where(counts > 0.0, counts, 1.0)
    means = sums / safe
    both = lax.dot_general(present, present, (((1,), (1,)), ((), ())),
                           preferred_element_type=jnp.float32)
    pm = _pairmask()
    ncomp = float(NSENS) * jnp.sum(pm * both)
    total = jnp.float32(0.0)
    for k in range(NGROUPS):
        d = jnp.abs(means - means[k:k + 1, :])
        s = jnp.sum(d, axis=1, keepdims=True)
        total = total + jnp.sum(s * pm[:, k:k + 1] * both[:, k:k + 1])
    result = jnp.where(
        ncomp > 0.0,
        LAMBDA_FAIRNESS * total / jnp.maximum(ncomp, 1.0),
        0.0)
    out_ref[0, 0] = result


@jax.jit
def kernel(action_probs, demo_gender, demo_age):
    xs = action_probs[:, :NSENS].reshape(R, 128)
    combo = (demo_gender + 2 * demo_age).astype(jnp.int8).reshape(R, 8)
    out = pl.pallas_call(
        _body,
        out_specs=pl.BlockSpec(memory_space=pltpu.SMEM),
        out_shape=jax.ShapeDtypeStruct((1, 1), jnp.float32),
    )(xs, combo)
    return out[0, 0]


# speedup vs baseline: 13.3015x; 2.8982x over previous
import jax
import jax.numpy as jnp
from jax import lax
from jax.experimental import pallas as pl
from jax.experimental.pallas import tpu as pltpu

BATCH = 16384
NUM_ACTIONS = 1000
NSENS = 16
NGROUPS = 10
LAMBDA_FAIRNESS = 0.01


def _pairmask():
    rj = lax.broadcasted_iota(jnp.int32, (16, 16), 0)
    ck = lax.broadcasted_iota(jnp.int32, (16, 16), 1)
    same = jnp.logical_or(
        jnp.logical_and(rj < 2, ck < 2),
        jnp.logical_and(jnp.logical_and(rj >= 2, rj < 10),
                        jnp.logical_and(ck >= 2, ck < 10)))
    return jnp.where(jnp.logical_and(rj > ck, same), 1.0, 0.0)


def _body(x_ref, c_ref, out_ref):
    XT = x_ref[...]
    cmb = c_ref[...]
    gi = lax.broadcasted_iota(jnp.int32, (16, BATCH), 0)
    is_gender = gi < 2
    oh_bool = jnp.logical_or(
        jnp.logical_and(is_gender, (cmb % 2) == gi),
        jnp.logical_and(
            jnp.logical_and(jnp.logical_not(is_gender), gi < NGROUPS),
            (cmb // 2) == (gi - 2)))
    MT = jnp.where(oh_bool, 1.0, 0.0)

    sums = lax.dot_general(MT, XT, (((1,), (1,)), ((), ())),
                           preferred_element_type=jnp.float32)
    counts = lax.dot_general(MT, jnp.ones((1, BATCH), jnp.float32),
                             (((1,), (1,)), ((), ())),
                             preferred_element_type=jnp.float32)

    present = jnp.where(counts > 0.0, 1.0, 0.0)
    safe = jnp.where(counts > 0.0, counts, 1.0)
    means = sums / safe
    both = lax.dot_general(present, present, (((1,), (1,)), ((), ())),
                           preferred_element_type=jnp.float32)
    pm = _pairmask()
    ncomp = float(NSENS) * jnp.sum(pm * both)
    total = jnp.float32(0.0)
    for k in range(NGROUPS):
        d = jnp.abs(means - means[k:k + 1, :])
        s = jnp.sum(d, axis=1, keepdims=True)
        total = total + jnp.sum(s * pm[:, k:k + 1] * both[:, k:k + 1])
    result = jnp.where(
        ncomp > 0.0,
        LAMBDA_FAIRNESS * total / jnp.maximum(ncomp, 1.0),
        0.0)
    out_ref[0, 0] = result


@jax.jit
def kernel(action_probs, demo_gender, demo_age):
    xt = action_probs[:, :NSENS].T
    combo = (demo_gender + 2 * demo_age).reshape(1, BATCH)
    out = pl.pallas_call(
        _body,
        out_specs=pl.BlockSpec(memory_space=pltpu.SMEM),
        out_shape=jax.ShapeDtypeStruct((1, 1), jnp.float32),
    )(xt, combo)
    return out[0, 0]


# speedup vs baseline: 17.6025x; 1.3233x over previous
import jax
import jax.numpy as jnp
from jax import lax
from jax.experimental import pallas as pl
from jax.experimental.pallas import tpu as pltpu

BATCH = 16384
NUM_ACTIONS = 1000
NSENS = 16
NGROUPS = 10
LAMBDA_FAIRNESS = 0.01


def _pairmask():
    rj = lax.broadcasted_iota(jnp.int32, (16, 16), 0)
    ck = lax.broadcasted_iota(jnp.int32, (16, 16), 1)
    same = jnp.logical_or(
        jnp.logical_and(rj < 2, ck < 2),
        jnp.logical_and(jnp.logical_and(rj >= 2, rj < 10),
                        jnp.logical_and(ck >= 2, ck < 10)))
    return jnp.where(jnp.logical_and(rj > ck, same), 1.0, 0.0)


def _body(x_ref, g_ref, a_ref, out_ref):
    XT = x_ref[...]
    gvec = g_ref[...]
    avec = a_ref[...]
    gi = lax.broadcasted_iota(jnp.int32, (16, BATCH), 0)
    is_gender = gi < 2
    oh_bool = jnp.logical_or(
        jnp.logical_and(is_gender, gvec == gi),
        jnp.logical_and(
            jnp.logical_and(jnp.logical_not(is_gender), gi < NGROUPS),
            avec == (gi - 2)))
    MT = jnp.where(oh_bool, 1.0, 0.0)

    sums = lax.dot_general(MT, XT, (((1,), (1,)), ((), ())),
                           preferred_element_type=jnp.float32)
    counts = lax.dot_general(MT, jnp.ones((1, BATCH), jnp.float32),
                             (((1,), (1,)), ((), ())),
                             preferred_element_type=jnp.float32)

    present = jnp.where(counts > 0.0, 1.0, 0.0)
    safe = jnp.where(counts > 0.0, counts, 1.0)
    means = sums / safe
    both = lax.dot_general(present, present, (((1,), (1,)), ((), ())),
                           preferred_element_type=jnp.float32)
    pm = _pairmask()
    ncomp = float(NSENS) * jnp.sum(pm * both)
    total = jnp.float32(0.0)
    for k in range(NGROUPS):
        d = jnp.abs(means - means[k:k + 1, :])
        s = jnp.sum(d, axis=1, keepdims=True)
        total = total + jnp.sum(s * pm[:, k:k + 1] * both[:, k:k + 1])
    result = jnp.where(
        ncomp > 0.0,
        LAMBDA_FAIRNESS * total / jnp.maximum(ncomp, 1.0),
        0.0)
    out_ref[0, 0] = result


@jax.jit
def kernel(action_probs, demo_gender, demo_age):
    xt = action_probs[:, :NSENS].T
    out = pl.pallas_call(
        _body,
        out_specs=pl.BlockSpec(memory_space=pltpu.SMEM),
        out_shape=jax.ShapeDtypeStruct((1, 1), jnp.float32),
    )(xt, demo_gender.reshape(1, BATCH), demo_age.reshape(1, BATCH))
    return out[0, 0]


# speedup vs baseline: 31.7366x; 1.8030x over previous
import jax
import jax.numpy as jnp
from jax import lax
from jax.experimental import pallas as pl
from jax.experimental.pallas import tpu as pltpu

BATCH = 16384
NUM_ACTIONS = 1000
NSENS = 16
NGROUPS = 10
LAMBDA_FAIRNESS = 0.01


def _pairmask():
    rj = lax.broadcasted_iota(jnp.int32, (16, 16), 0)
    ck = lax.broadcasted_iota(jnp.int32, (16, 16), 1)
    same = jnp.logical_or(
        jnp.logical_and(rj < 2, ck < 2),
        jnp.logical_and(jnp.logical_and(rj >= 2, rj < 10),
                        jnp.logical_and(ck >= 2, ck < 10)))
    return jnp.where(jnp.logical_and(rj > ck, same), 1.0, 0.0)


def _body(x_ref, g_ref, a_ref, out_ref):
    XT = x_ref[...]
    gvec = g_ref[...]
    avec = a_ref[...]
    gi = lax.broadcasted_iota(jnp.int32, (16, BATCH), 0)
    is_gender = gi < 2
    oh_bool = jnp.logical_or(
        jnp.logical_and(is_gender, gvec == gi),
        jnp.logical_and(
            jnp.logical_and(jnp.logical_not(is_gender), gi < NGROUPS),
            avec == (gi - 2)))
    MT = jnp.where(oh_bool, 1.0, 0.0)

    sums = lax.dot_general(MT, XT, (((1,), (1,)), ((), ())),
                           preferred_element_type=jnp.float32)
    counts = lax.dot_general(MT, jnp.ones((1, BATCH), jnp.float32),
                             (((1,), (1,)), ((), ())),
                             preferred_element_type=jnp.float32)

    present = jnp.where(counts > 0.0, 1.0, 0.0)
    safe = jnp.where(counts > 0.0, counts, 1.0)
    means = sums / safe
    both = lax.dot_general(present, present, (((1,), (1,)), ((), ())),
                           preferred_element_type=jnp.float32)
    pm = _pairmask()
    ncomp = float(NSENS) * jnp.sum(pm * both)
    total = jnp.float32(0.0)
    for k in range(NGROUPS):
        d = jnp.abs(means - means[k:k + 1, :])
        s = jnp.sum(d, axis=1, keepdims=True)
        total = total + jnp.sum(s * pm[:, k:k + 1] * both[:, k:k + 1])
    result = jnp.where(
        ncomp > 0.0,
        LAMBDA_FAIRNESS * total / jnp.maximum(ncomp, 1.0),
        0.0)
    out_ref[0, 0] = result


@jax.jit
def kernel(action_probs, demo_gender, demo_age):
    xt = action_probs[:, :NSENS].T
    out = pl.pallas_call(
        _body,
        out_specs=pl.BlockSpec(memory_space=pltpu.SMEM),
        out_shape=jax.ShapeDtypeStruct((1, 1), jnp.float32),
        compiler_params=pltpu.CompilerParams(
            allow_input_fusion=[True, False, False]),
    )(xt, demo_gender.reshape(1, BATCH), demo_age.reshape(1, BATCH))
    return out[0, 0]


# speedup vs baseline: 32.3504x; 1.0193x over previous
import jax
import jax.numpy as jnp
from jax import lax
from jax.experimental import pallas as pl
from jax.experimental.pallas import tpu as pltpu

BATCH = 16384
NUM_ACTIONS = 1000
NSENS = 16
NGROUPS = 10
LAMBDA_FAIRNESS = 0.01


def _pairmask():
    rj = lax.broadcasted_iota(jnp.int32, (16, 16), 0)
    ck = lax.broadcasted_iota(jnp.int32, (16, 16), 1)
    same = jnp.logical_or(
        jnp.logical_and(rj < 2, ck < 2),
        jnp.logical_and(jnp.logical_and(rj >= 2, rj < 10),
                        jnp.logical_and(ck >= 2, ck < 10)))
    return jnp.where(jnp.logical_and(rj > ck, same), 1.0, 0.0)


def _body(x_ref, g_ref, a_ref, out_ref):
    XT = x_ref[...]
    gvec = g_ref[...]
    avec = a_ref[...]
    gi = lax.broadcasted_iota(jnp.int32, (16, BATCH), 0)
    is_gender = gi < 2
    oh_bool = jnp.logical_or(
        jnp.logical_and(is_gender, gvec == gi),
        jnp.logical_and(
            jnp.logical_and(jnp.logical_not(is_gender), gi < NGROUPS),
            avec == (gi - 2)))
    MT = jnp.where(oh_bool, 1.0, 0.0)

    sums = lax.dot_general(MT, XT, (((1,), (1,)), ((), ())),
                           preferred_element_type=jnp.float32)
    counts = lax.dot_general(MT, jnp.ones((1, BATCH), jnp.float32),
                             (((1,), (1,)), ((), ())),
                             preferred_element_type=jnp.float32)

    present = jnp.where(counts > 0.0, 1.0, 0.0)
    safe = jnp.where(counts > 0.0, counts, 1.0)
    means = sums / safe
    both = lax.dot_general(present, present, (((1,), (1,)), ((), ())),
                           preferred_element_type=jnp.float32)
    pm = _pairmask()
    ncomp = float(NSENS) * jnp.sum(pm * both)
    total = jnp.float32(0.0)
    for k in range(NGROUPS):
        d = jnp.abs(means - means[k:k + 1, :])
        s = jnp.sum(d, axis=1, keepdims=True)
        total = total + jnp.sum(s * pm[:, k:k + 1] * both[:, k:k + 1])
    result = jnp.where(
        ncomp > 0.0,
        LAMBDA_FAIRNESS * total / jnp.maximum(ncomp, 1.0),
        0.0)
    out_ref[0, 0] = result


@jax.jit
def kernel(action_probs, demo_gender, demo_age):
    xt = action_probs[:, :NSENS].T
    out = pl.pallas_call(
        _body,
        out_specs=pl.BlockSpec(memory_space=pltpu.SMEM),
        out_shape=jax.ShapeDtypeStruct((1, 1), jnp.float32),
        compiler_params=pltpu.CompilerParams(
            allow_input_fusion=[True, True, True]),
    )(xt, demo_gender.reshape(1, BATCH), demo_age.reshape(1, BATCH))
    return out[0, 0]
